# baseline (speedup 1.0000x reference)
import functools

import jax
import jax.numpy as jnp
from jax import lax
from jax.experimental import pallas as pl
from jax.experimental.pallas import tpu as pltpu
from jax.experimental.pallas import tpu_sc as plsc

_NC = 2
_NS = 16
_NW = _NC * _NS
_CHUNK = 128
_DW = 8
_G = 64


def _make_deg_kernel(NPAD, NCHUNK):
    rpt = NPAD // _NS
    mesh = plsc.VectorSubcoreMesh(core_axis_name="c", subcore_axis_name="s")

    @functools.partial(
        pl.kernel,
        out_type=jax.ShapeDtypeStruct((_NC, NPAD, _DW), jnp.float32),
        mesh=mesh,
        scratch_types=[
            pltpu.VMEM((NCHUNK, _CHUNK), jnp.int32),
            pltpu.VMEM((_CHUNK, _DW), jnp.float32),
            pltpu.VMEM_SHARED((NPAD, _DW), jnp.float32),
            pltpu.SemaphoreType.DMA,
        ],
        compiler_params=pltpu.CompilerParams(use_tc_tiling_on_sc=False),
    )
    def deg_kernel(dst_hbm, ones_hbm, zeros_hbm, out_hbm, didx, ones_v, acc,
                   sem):
        cid = lax.axis_index("c")
        sid = lax.axis_index("s")
        wid = sid * _NC + cid
        r0 = sid * rpt
        pltpu.sync_copy(zeros_hbm, acc.at[pl.ds(r0, rpt)])
        pltpu.sync_copy(ones_hbm, ones_v)
        pltpu.sync_copy(dst_hbm.at[wid], didx)
        plsc.subcore_barrier()

        def fire(j, carry):
            pltpu.async_copy(ones_v, acc.at[didx.at[j]], sem, add=True)
            return carry

        lax.fori_loop(0, NCHUNK, fire, 0)

        def drain(j, carry):
            pltpu.make_async_copy(ones_v, acc.at[didx.at[j]], sem).wait()
            return carry

        lax.fori_loop(0, NCHUNK, drain, 0)
        plsc.subcore_barrier()
        pltpu.sync_copy(acc.at[pl.ds(r0, rpt)],
                        out_hbm.at[cid, pl.ds(r0, rpt)])

    return deg_kernel


_NB = 4


def _make_msg_kernel(NPAD, H, NCHUNK):
    rpt = NPAD // _NS
    mesh = plsc.VectorSubcoreMesh(core_axis_name="c", subcore_axis_name="s")

    @functools.partial(
        pl.kernel,
        out_type=jax.ShapeDtypeStruct((_NC, NPAD, H), jnp.float32),
        mesh=mesh,
        scratch_types=[
            pltpu.VMEM((NCHUNK, _CHUNK), jnp.int32),
            pltpu.VMEM((NCHUNK, _CHUNK), jnp.int32),
            [pltpu.VMEM((_CHUNK, H), jnp.float32)] * _NB,
            [pltpu.SemaphoreType.DMA] * _NB,
            pltpu.SemaphoreType.DMA,
            pltpu.VMEM_SHARED((NPAD, H), jnp.float32),
        ],
        compiler_params=pltpu.CompilerParams(use_tc_tiling_on_sc=False),
    )
    def msg_kernel(hs_hbm, src_hbm, dst_hbm, zeros_hbm, out_hbm,
                   sidx, didx, rows, sems, gsem, acc):
        cid = lax.axis_index("c")
        sid = lax.axis_index("s")
        wid = sid * _NC + cid
        r0 = sid * rpt

        pltpu.sync_copy(src_hbm.at[wid], sidx)
        pltpu.sync_copy(dst_hbm.at[wid], didx)

        @pl.when(cid == 0)
        def _():
            pltpu.sync_copy(hs_hbm.at[pl.ds(r0, rpt)],
                            acc.at[pl.ds(r0, rpt)])

        @pl.when(cid != 0)
        def _():
            pltpu.sync_copy(zeros_hbm, acc.at[pl.ds(r0, rpt)])

        plsc.subcore_barrier()

        def wait_scatter(chunk, b):
            pltpu.make_async_copy(rows[b], acc.at[didx.at[chunk]],
                                  sems[b]).wait()

        def body(j, carry):
            for b in range(_NB):
                chunk = j + b

                @pl.when(chunk >= _NB)
                def _():
                    wait_scatter(chunk - _NB, b)

                pltpu.async_copy(hs_hbm.at[sidx.at[chunk]],
                                 rows[b], gsem).wait()
                pltpu.async_copy(rows[b], acc.at[didx.at[chunk]],
                                 sems[b], add=True)

            return carry

        lax.fori_loop(0, NCHUNK // _NB, lambda i, c: body(i * _NB, c), 0,
                      unroll=False)
        for b in range(_NB):
            wait_scatter(NCHUNK - _NB + b, b)
        plsc.subcore_barrier()
        pltpu.sync_copy(acc.at[pl.ds(r0, rpt)],
                        out_hbm.at[cid, pl.ds(r0, rpt)])

    return msg_kernel


def _dis_from_deg(d0_ref, d1_ref):
    deg = (d0_ref[...] + d1_ref[...])[:, 0:1] + 1.0
    return lax.rsqrt(deg)


def _tc_first(x_ref, w_ref, d0_ref, d1_ref, o_ref):
    dis = _dis_from_deg(d0_ref, d1_ref)
    h = jnp.dot(x_ref[...], w_ref[...], preferred_element_type=jnp.float32)
    o_ref[...] = h * dis


def _tc_mid(a0_ref, a1_ref, d0_ref, d1_ref, w_ref, b_ref, o_ref):
    dis = _dis_from_deg(d0_ref, d1_ref)
    pre = (a0_ref[...] + a1_ref[...]) * dis + b_ref[...]
    act = jnp.maximum(pre, 0.0)
    h = jnp.dot(act, w_ref[...], preferred_element_type=jnp.float32)
    o_ref[...] = h * dis


def _tc_final(a0_ref, a1_ref, d0_ref, d1_ref, b_ref, batch_ref,
              lw_ref, lb_ref, o_ref):
    npad = a0_ref.shape[0]
    dis = _dis_from_deg(d0_ref, d1_ref)
    h3 = (a0_ref[...] + a1_ref[...]) * dis + b_ref[...]
    seg = lax.broadcasted_iota(jnp.int32, (1, _G), 1)
    m = (batch_ref[...] == seg).astype(jnp.float32)
    psum = lax.dot_general(m, h3, (((0,), (0,)), ((), ())),
                           preferred_element_type=jnp.float32)
    cnt = lax.dot_general(m, jnp.ones((npad, 1), jnp.float32),
                          (((0,), (0,)), ((), ())),
                          preferred_element_type=jnp.float32)
    pooled = psum / jnp.maximum(cnt, 1.0)
    o_ref[...] = (jnp.dot(pooled, lw_ref[...],
                          preferred_element_type=jnp.float32) + lb_ref[...])


def kernel(x, edge_index, batch, W1, b1, W2, b2, W3, b3, lin_W, lin_b):
    N, F = x.shape
    H = W1.shape[1]
    C = lin_W.shape[1]
    E = edge_index.shape[1]

    NCHUNK = -(-E // (_NW * _CHUNK))
    NCHUNK = -(-NCHUNK // _NB) * _NB
    EPAD = _NW * NCHUNK * _CHUNK
    NPAD = -(-(N + 1) // (_NS * 16)) * (_NS * 16)
    rpt = NPAD // _NS

    src = edge_index[0]
    dst = edge_index[1]
    epad = EPAD - E
    srcp = jnp.concatenate(
        [src, jnp.zeros((epad,), jnp.int32)]).reshape(_NW, NCHUNK, _CHUNK)
    dstp = jnp.concatenate(
        [dst, jnp.full((epad,), N, jnp.int32)]).reshape(_NW, NCHUNK, _CHUNK)
    xp = jnp.zeros((NPAD, F), jnp.float32).at[:N].set(x)
    zeros_h = jnp.zeros((rpt, H), jnp.float32)
    zeros_d = jnp.zeros((rpt, _DW), jnp.float32)
    ones_d = jnp.ones((_CHUNK, _DW), jnp.float32)
    batchp = jnp.concatenate(
        [batch, jnp.full((NPAD - N,), _G, jnp.int32)]).reshape(NPAD, 1)
    b1r = b1.reshape(1, H)
    b2r = b2.reshape(1, H)
    b3r = b3.reshape(1, H)
    lbr = lin_b.reshape(1, C)

    deg_kernel = _make_deg_kernel(NPAD, NCHUNK)
    msg_kernel = _make_msg_kernel(NPAD, H, NCHUNK)

    degp = deg_kernel(dstp, ones_d, zeros_d)
    d0, d1 = degp[0], degp[1]

    h1s = pl.pallas_call(
        _tc_first,
        out_shape=jax.ShapeDtypeStruct((NPAD, H), jnp.float32),
    )(xp, W1, d0, d1)

    acc1 = msg_kernel(h1s, srcp, dstp, zeros_h)
    h2s = pl.pallas_call(
        _tc_mid,
        out_shape=jax.ShapeDtypeStruct((NPAD, H), jnp.float32),
    )(acc1[0], acc1[1], d0, d1, W2, b1r)

    acc2 = msg_kernel(h2s, srcp, dstp, zeros_h)
    h3s = pl.pallas_call(
        _tc_mid,
        out_shape=jax.ShapeDtypeStruct((NPAD, H), jnp.float32),
    )(acc2[0], acc2[1], d0, d1, W3, b2r)

    acc3 = msg_kernel(h3s, srcp, dstp, zeros_h)
    logits = pl.pallas_call(
        _tc_final,
        out_shape=jax.ShapeDtypeStruct((_G, C), jnp.float32),
    )(acc3[0], acc3[1], d0, d1, b3r, batchp, lin_W, lbr)

    return logits


# speedup vs baseline: 1.0122x; 1.0122x over previous
import functools

import jax
import jax.numpy as jnp
from jax import lax
from jax.experimental import pallas as pl
from jax.experimental.pallas import tpu as pltpu
from jax.experimental.pallas import tpu_sc as plsc

_NC = 2
_NS = 16
_NW = _NC * _NS
_CHUNK = 512
_DW = 8
_G = 64


def _make_deg_kernel(NPAD, NCHUNK):
    rpt = NPAD // _NS
    mesh = plsc.VectorSubcoreMesh(core_axis_name="c", subcore_axis_name="s")

    @functools.partial(
        pl.kernel,
        out_type=jax.ShapeDtypeStruct((_NC, NPAD, _DW), jnp.float32),
        mesh=mesh,
        scratch_types=[
            pltpu.VMEM((NCHUNK, _CHUNK), jnp.int32),
            pltpu.VMEM((_CHUNK, _DW), jnp.float32),
            pltpu.VMEM_SHARED((NPAD, _DW), jnp.float32),
            pltpu.SemaphoreType.DMA,
        ],
        compiler_params=pltpu.CompilerParams(use_tc_tiling_on_sc=False),
    )
    def deg_kernel(dst_hbm, ones_hbm, zeros_hbm, out_hbm, didx, ones_v, acc,
                   sem):
        cid = lax.axis_index("c")
        sid = lax.axis_index("s")
        wid = sid * _NC + cid
        r0 = sid * rpt
        pltpu.sync_copy(zeros_hbm, acc.at[pl.ds(r0, rpt)])
        pltpu.sync_copy(ones_hbm, ones_v)
        pltpu.sync_copy(dst_hbm.at[wid], didx)
        plsc.subcore_barrier()

        def fire(j, carry):
            pltpu.async_copy(ones_v, acc.at[didx.at[j]], sem, add=True)
            return carry

        lax.fori_loop(0, NCHUNK, fire, 0)

        def drain(j, carry):
            pltpu.make_async_copy(ones_v, acc.at[didx.at[j]], sem).wait()
            return carry

        lax.fori_loop(0, NCHUNK, drain, 0)
        plsc.subcore_barrier()
        pltpu.sync_copy(acc.at[pl.ds(r0, rpt)],
                        out_hbm.at[cid, pl.ds(r0, rpt)])

    return deg_kernel


_NB = 4


def _make_msg_kernel(NPAD, H, NCHUNK):
    rpt = NPAD // _NS
    mesh = plsc.VectorSubcoreMesh(core_axis_name="c", subcore_axis_name="s")

    @functools.partial(
        pl.kernel,
        out_type=jax.ShapeDtypeStruct((_NC, NPAD, H), jnp.float32),
        mesh=mesh,
        scratch_types=[
            pltpu.VMEM((NCHUNK, _CHUNK), jnp.int32),
            pltpu.VMEM((NCHUNK, _CHUNK), jnp.int32),
            pltpu.VMEM((_CHUNK, H), jnp.float32),
            pltpu.SemaphoreType.DMA,
            pltpu.VMEM_SHARED((NPAD, H), jnp.float32),
        ],
        compiler_params=pltpu.CompilerParams(use_tc_tiling_on_sc=False),
    )
    def msg_kernel(hs_hbm, src_hbm, dst_hbm, zeros_hbm, out_hbm,
                   sidx, didx, rows, gsem, acc):
        cid = lax.axis_index("c")
        sid = lax.axis_index("s")
        wid = sid * _NC + cid
        r0 = sid * rpt

        pltpu.sync_copy(src_hbm.at[wid], sidx)
        pltpu.sync_copy(dst_hbm.at[wid], didx)

        @pl.when(cid == 0)
        def _():
            pltpu.sync_copy(hs_hbm.at[pl.ds(r0, rpt)],
                            acc.at[pl.ds(r0, rpt)])

        @pl.when(cid != 0)
        def _():
            pltpu.sync_copy(zeros_hbm, acc.at[pl.ds(r0, rpt)])

        plsc.subcore_barrier()

        def body(j, carry):
            pltpu.async_copy(hs_hbm.at[sidx.at[j]], rows, gsem).wait()
            pltpu.sync_copy(rows, acc.at[didx.at[j]], add=True)
            return carry

        lax.fori_loop(0, NCHUNK, body, 0, unroll=False)
        plsc.subcore_barrier()
        pltpu.sync_copy(acc.at[pl.ds(r0, rpt)],
                        out_hbm.at[cid, pl.ds(r0, rpt)])

    return msg_kernel


def _dis_from_deg(d0_ref, d1_ref):
    deg = (d0_ref[...] + d1_ref[...])[:, 0:1] + 1.0
    return lax.rsqrt(deg)


def _tc_first(x_ref, w_ref, d0_ref, d1_ref, o_ref):
    dis = _dis_from_deg(d0_ref, d1_ref)
    h = jnp.dot(x_ref[...], w_ref[...], preferred_element_type=jnp.float32)
    o_ref[...] = h * dis


def _tc_mid(a0_ref, a1_ref, d0_ref, d1_ref, w_ref, b_ref, o_ref):
    dis = _dis_from_deg(d0_ref, d1_ref)
    pre = (a0_ref[...] + a1_ref[...]) * dis + b_ref[...]
    act = jnp.maximum(pre, 0.0)
    h = jnp.dot(act, w_ref[...], preferred_element_type=jnp.float32)
    o_ref[...] = h * dis


def _tc_final(a0_ref, a1_ref, d0_ref, d1_ref, b_ref, batch_ref,
              lw_ref, lb_ref, o_ref):
    npad = a0_ref.shape[0]
    dis = _dis_from_deg(d0_ref, d1_ref)
    h3 = (a0_ref[...] + a1_ref[...]) * dis + b_ref[...]
    seg = lax.broadcasted_iota(jnp.int32, (1, _G), 1)
    m = (batch_ref[...] == seg).astype(jnp.float32)
    psum = lax.dot_general(m, h3, (((0,), (0,)), ((), ())),
                           preferred_element_type=jnp.float32)
    cnt = lax.dot_general(m, jnp.ones((npad, 1), jnp.float32),
                          (((0,), (0,)), ((), ())),
                          preferred_element_type=jnp.float32)
    pooled = psum / jnp.maximum(cnt, 1.0)
    o_ref[...] = (jnp.dot(pooled, lw_ref[...],
                          preferred_element_type=jnp.float32) + lb_ref[...])


def kernel(x, edge_index, batch, W1, b1, W2, b2, W3, b3, lin_W, lin_b):
    N, F = x.shape
    H = W1.shape[1]
    C = lin_W.shape[1]
    E = edge_index.shape[1]

    NCHUNK = -(-E // (_NW * _CHUNK))
    NCHUNK = -(-NCHUNK // _NB) * _NB
    EPAD = _NW * NCHUNK * _CHUNK
    NPAD = -(-(N + 1) // (_NS * 16)) * (_NS * 16)
    rpt = NPAD // _NS

    src = edge_index[0]
    dst = edge_index[1]
    epad = EPAD - E
    srcp = jnp.concatenate(
        [src, jnp.zeros((epad,), jnp.int32)]).reshape(_NW, NCHUNK, _CHUNK)
    dstp = jnp.concatenate(
        [dst, jnp.full((epad,), N, jnp.int32)]).reshape(_NW, NCHUNK, _CHUNK)
    xp = jnp.zeros((NPAD, F), jnp.float32).at[:N].set(x)
    zeros_h = jnp.zeros((rpt, H), jnp.float32)
    zeros_d = jnp.zeros((rpt, _DW), jnp.float32)
    ones_d = jnp.ones((_CHUNK, _DW), jnp.float32)
    batchp = jnp.concatenate(
        [batch, jnp.full((NPAD - N,), _G, jnp.int32)]).reshape(NPAD, 1)
    b1r = b1.reshape(1, H)
    b2r = b2.reshape(1, H)
    b3r = b3.reshape(1, H)
    lbr = lin_b.reshape(1, C)

    deg_kernel = _make_deg_kernel(NPAD, NCHUNK)
    msg_kernel = _make_msg_kernel(NPAD, H, NCHUNK)

    degp = deg_kernel(dstp, ones_d, zeros_d)
    d0, d1 = degp[0], degp[1]

    h1s = pl.pallas_call(
        _tc_first,
        out_shape=jax.ShapeDtypeStruct((NPAD, H), jnp.float32),
    )(xp, W1, d0, d1)

    acc1 = msg_kernel(h1s, srcp, dstp, zeros_h)
    h2s = pl.pallas_call(
        _tc_mid,
        out_shape=jax.ShapeDtypeStruct((NPAD, H), jnp.float32),
    )(acc1[0], acc1[1], d0, d1, W2, b1r)

    acc2 = msg_kernel(h2s, srcp, dstp, zeros_h)
    h3s = pl.pallas_call(
        _tc_mid,
        out_shape=jax.ShapeDtypeStruct((NPAD, H), jnp.float32),
    )(acc2[0], acc2[1], d0, d1, W3, b2r)

    acc3 = msg_kernel(h3s, srcp, dstp, zeros_h)
    logits = pl.pallas_call(
        _tc_final,
        out_shape=jax.ShapeDtypeStruct((_G, C), jnp.float32),
    )(acc3[0], acc3[1], d0, d1, b3r, batchp, lin_W, lbr)

    return logits


# speedup vs baseline: 1.7764x; 1.7549x over previous
import functools

import jax
import jax.numpy as jnp
from jax import lax
from jax.experimental import pallas as pl
from jax.experimental.pallas import tpu as pltpu
from jax.experimental.pallas import tpu_sc as plsc

_NC = 2
_NS = 16
_NW = _NC * _NS
_CHUNK = 128
_DW = 8
_G = 64


def _make_deg_kernel(NPAD, NCHUNK):
    rpt = NPAD // _NS
    mesh = plsc.VectorSubcoreMesh(core_axis_name="c", subcore_axis_name="s")

    @functools.partial(
        pl.kernel,
        out_type=jax.ShapeDtypeStruct((_NC, NPAD, _DW), jnp.float32),
        mesh=mesh,
        scratch_types=[
            pltpu.VMEM((NCHUNK, _CHUNK), jnp.int32),
            pltpu.VMEM((_CHUNK, _DW), jnp.float32),
            pltpu.VMEM_SHARED((NPAD, _DW), jnp.float32),
            pltpu.SemaphoreType.DMA,
        ],
        compiler_params=pltpu.CompilerParams(use_tc_tiling_on_sc=False),
    )
    def deg_kernel(dst_hbm, ones_hbm, zeros_hbm, out_hbm, didx, ones_v, acc,
                   sem):
        cid = lax.axis_index("c")
        sid = lax.axis_index("s")
        wid = sid * _NC + cid
        r0 = sid * rpt
        pltpu.sync_copy(zeros_hbm, acc.at[pl.ds(r0, rpt)])
        pltpu.sync_copy(ones_hbm, ones_v)
        pltpu.sync_copy(dst_hbm.at[wid], didx)
        plsc.subcore_barrier()

        def fire(j, carry):
            pltpu.async_copy(ones_v, acc.at[didx.at[j]], sem, add=True)
            return carry

        lax.fori_loop(0, NCHUNK, fire, 0)

        def drain(j, carry):
            pltpu.make_async_copy(ones_v, acc.at[didx.at[j]], sem).wait()
            return carry

        lax.fori_loop(0, NCHUNK, drain, 0)
        plsc.subcore_barrier()
        pltpu.sync_copy(acc.at[pl.ds(r0, rpt)],
                        out_hbm.at[cid, pl.ds(r0, rpt)])

    return deg_kernel


_NB = 4


def _make_msg_kernel(NPAD, H, NCHUNK):
    rpt = NPAD // _NS
    mesh = plsc.VectorSubcoreMesh(core_axis_name="c", subcore_axis_name="s")

    @functools.partial(
        pl.kernel,
        out_type=jax.ShapeDtypeStruct((_NC, NPAD, H), jnp.float32),
        mesh=mesh,
        scratch_types=[
            pltpu.VMEM((NCHUNK, _CHUNK), jnp.int32),
            pltpu.VMEM((NCHUNK, _CHUNK), jnp.int32),
            pltpu.VMEM((_CHUNK, H), jnp.float32),
            pltpu.SemaphoreType.DMA,
            pltpu.VMEM_SHARED((NPAD, H), jnp.float32),
            pltpu.VMEM_SHARED((NPAD, H), jnp.float32),
        ],
        compiler_params=pltpu.CompilerParams(use_tc_tiling_on_sc=False),
    )
    def msg_kernel(hs_hbm, src_hbm, dst_hbm, zeros_hbm, out_hbm,
                   sidx, didx, rows, gsem, acc, hsc):
        cid = lax.axis_index("c")
        sid = lax.axis_index("s")
        wid = sid * _NC + cid
        r0 = sid * rpt

        pltpu.sync_copy(src_hbm.at[wid], sidx)
        pltpu.sync_copy(dst_hbm.at[wid], didx)

        pltpu.sync_copy(hs_hbm.at[pl.ds(r0, rpt)], hsc.at[pl.ds(r0, rpt)])

        @pl.when(cid == 0)
        def _():
            pltpu.sync_copy(hs_hbm.at[pl.ds(r0, rpt)],
                            acc.at[pl.ds(r0, rpt)])

        @pl.when(cid != 0)
        def _():
            pltpu.sync_copy(zeros_hbm, acc.at[pl.ds(r0, rpt)])

        plsc.subcore_barrier()

        def body(j, carry):
            pltpu.async_copy(hsc.at[sidx.at[j]], rows, gsem).wait()
            pltpu.sync_copy(rows, acc.at[didx.at[j]], add=True)
            return carry

        lax.fori_loop(0, NCHUNK, body, 0, unroll=False)
        plsc.subcore_barrier()
        pltpu.sync_copy(acc.at[pl.ds(r0, rpt)],
                        out_hbm.at[cid, pl.ds(r0, rpt)])

    return msg_kernel


def _dis_from_deg(d0_ref, d1_ref):
    deg = (d0_ref[...] + d1_ref[...])[:, 0:1] + 1.0
    return lax.rsqrt(deg)


def _tc_first(x_ref, w_ref, d0_ref, d1_ref, o_ref):
    dis = _dis_from_deg(d0_ref, d1_ref)
    h = jnp.dot(x_ref[...], w_ref[...], preferred_element_type=jnp.float32)
    o_ref[...] = h * dis


def _tc_mid(a0_ref, a1_ref, d0_ref, d1_ref, w_ref, b_ref, o_ref):
    dis = _dis_from_deg(d0_ref, d1_ref)
    pre = (a0_ref[...] + a1_ref[...]) * dis + b_ref[...]
    act = jnp.maximum(pre, 0.0)
    h = jnp.dot(act, w_ref[...], preferred_element_type=jnp.float32)
    o_ref[...] = h * dis


def _tc_final(a0_ref, a1_ref, d0_ref, d1_ref, b_ref, batch_ref,
              lw_ref, lb_ref, o_ref):
    npad = a0_ref.shape[0]
    dis = _dis_from_deg(d0_ref, d1_ref)
    h3 = (a0_ref[...] + a1_ref[...]) * dis + b_ref[...]
    seg = lax.broadcasted_iota(jnp.int32, (1, _G), 1)
    m = (batch_ref[...] == seg).astype(jnp.float32)
    psum = lax.dot_general(m, h3, (((0,), (0,)), ((), ())),
                           preferred_element_type=jnp.float32)
    cnt = lax.dot_general(m, jnp.ones((npad, 1), jnp.float32),
                          (((0,), (0,)), ((), ())),
                          preferred_element_type=jnp.float32)
    pooled = psum / jnp.maximum(cnt, 1.0)
    o_ref[...] = (jnp.dot(pooled, lw_ref[...],
                          preferred_element_type=jnp.float32) + lb_ref[...])


def kernel(x, edge_index, batch, W1, b1, W2, b2, W3, b3, lin_W, lin_b):
    N, F = x.shape
    H = W1.shape[1]
    C = lin_W.shape[1]
    E = edge_index.shape[1]

    NCHUNK = -(-E // (_NW * _CHUNK))
    NCHUNK = -(-NCHUNK // _NB) * _NB
    EPAD = _NW * NCHUNK * _CHUNK
    NPAD = -(-(N + 1) // (_NS * 16)) * (_NS * 16)
    rpt = NPAD // _NS

    src = edge_index[0]
    dst = edge_index[1]
    epad = EPAD - E
    srcp = jnp.concatenate(
        [src, jnp.zeros((epad,), jnp.int32)]).reshape(_NW, NCHUNK, _CHUNK)
    dstp = jnp.concatenate(
        [dst, jnp.full((epad,), N, jnp.int32)]).reshape(_NW, NCHUNK, _CHUNK)
    xp = jnp.zeros((NPAD, F), jnp.float32).at[:N].set(x)
    zeros_h = jnp.zeros((rpt, H), jnp.float32)
    zeros_d = jnp.zeros((rpt, _DW), jnp.float32)
    ones_d = jnp.ones((_CHUNK, _DW), jnp.float32)
    batchp = jnp.concatenate(
        [batch, jnp.full((NPAD - N,), _G, jnp.int32)]).reshape(NPAD, 1)
    b1r = b1.reshape(1, H)
    b2r = b2.reshape(1, H)
    b3r = b3.reshape(1, H)
    lbr = lin_b.reshape(1, C)

    deg_kernel = _make_deg_kernel(NPAD, NCHUNK)
    msg_kernel = _make_msg_kernel(NPAD, H, NCHUNK)

    degp = deg_kernel(dstp, ones_d, zeros_d)
    d0, d1 = degp[0], degp[1]

    h1s = pl.pallas_call(
        _tc_first,
        out_shape=jax.ShapeDtypeStruct((NPAD, H), jnp.float32),
    )(xp, W1, d0, d1)

    acc1 = msg_kernel(h1s, srcp, dstp, zeros_h)
    h2s = pl.pallas_call(
        _tc_mid,
        out_shape=jax.ShapeDtypeStruct((NPAD, H), jnp.float32),
    )(acc1[0], acc1[1], d0, d1, W2, b1r)

    acc2 = msg_kernel(h2s, srcp, dstp, zeros_h)
    h3s = pl.pallas_call(
        _tc_mid,
        out_shape=jax.ShapeDtypeStruct((NPAD, H), jnp.float32),
    )(acc2[0], acc2[1], d0, d1, W3, b2r)

    acc3 = msg_kernel(h3s, srcp, dstp, zeros_h)
    logits = pl.pallas_call(
        _tc_final,
        out_shape=jax.ShapeDtypeStruct((_G, C), jnp.float32),
    )(acc3[0], acc3[1], d0, d1, b3r, batchp, lin_W, lbr)

    return logits


# speedup vs baseline: 1.8070x; 1.0173x over previous
import functools

import jax
import jax.numpy as jnp
from jax import lax
from jax.experimental import pallas as pl
from jax.experimental.pallas import tpu as pltpu
from jax.experimental.pallas import tpu_sc as plsc

_NC = 2
_NS = 16
_NW = _NC * _NS
_CHUNK = 256
_DW = 8
_G = 64


def _make_deg_kernel(NPAD, NCHUNK):
    rpt = NPAD // _NS
    mesh = plsc.VectorSubcoreMesh(core_axis_name="c", subcore_axis_name="s")

    @functools.partial(
        pl.kernel,
        out_type=jax.ShapeDtypeStruct((_NC, NPAD, _DW), jnp.float32),
        mesh=mesh,
        scratch_types=[
            pltpu.VMEM((NCHUNK, _CHUNK), jnp.int32),
            pltpu.VMEM((_CHUNK, _DW), jnp.float32),
            pltpu.VMEM_SHARED((NPAD, _DW), jnp.float32),
            pltpu.SemaphoreType.DMA,
        ],
        compiler_params=pltpu.CompilerParams(use_tc_tiling_on_sc=False),
    )
    def deg_kernel(dst_hbm, ones_hbm, zeros_hbm, out_hbm, didx, ones_v, acc,
                   sem):
        cid = lax.axis_index("c")
        sid = lax.axis_index("s")
        wid = sid * _NC + cid
        r0 = sid * rpt
        pltpu.sync_copy(zeros_hbm, acc.at[pl.ds(r0, rpt)])
        pltpu.sync_copy(ones_hbm, ones_v)
        pltpu.sync_copy(dst_hbm.at[wid], didx)
        plsc.subcore_barrier()

        def fire(j, carry):
            pltpu.async_copy(ones_v, acc.at[didx.at[j]], sem, add=True)
            return carry

        lax.fori_loop(0, NCHUNK, fire, 0)

        def drain(j, carry):
            pltpu.make_async_copy(ones_v, acc.at[didx.at[j]], sem).wait()
            return carry

        lax.fori_loop(0, NCHUNK, drain, 0)
        plsc.subcore_barrier()
        pltpu.sync_copy(acc.at[pl.ds(r0, rpt)],
                        out_hbm.at[cid, pl.ds(r0, rpt)])

    return deg_kernel


_NB = 4


def _make_msg_kernel(NPAD, H, NCHUNK):
    rpt = NPAD // _NS
    mesh = plsc.VectorSubcoreMesh(core_axis_name="c", subcore_axis_name="s")

    @functools.partial(
        pl.kernel,
        out_type=jax.ShapeDtypeStruct((_NC, NPAD, H), jnp.float32),
        mesh=mesh,
        scratch_types=[
            pltpu.VMEM((NCHUNK, _CHUNK), jnp.int32),
            pltpu.VMEM((NCHUNK, _CHUNK), jnp.int32),
            pltpu.VMEM((_CHUNK, H), jnp.float32),
            pltpu.SemaphoreType.DMA,
            pltpu.VMEM_SHARED((NPAD, H), jnp.float32),
            pltpu.VMEM_SHARED((NPAD, H), jnp.float32),
        ],
        compiler_params=pltpu.CompilerParams(use_tc_tiling_on_sc=False),
    )
    def msg_kernel(hs_hbm, src_hbm, dst_hbm, zeros_hbm, out_hbm,
                   sidx, didx, rows, gsem, acc, hsc):
        cid = lax.axis_index("c")
        sid = lax.axis_index("s")
        wid = sid * _NC + cid
        r0 = sid * rpt

        pltpu.sync_copy(src_hbm.at[wid], sidx)
        pltpu.sync_copy(dst_hbm.at[wid], didx)

        pltpu.sync_copy(hs_hbm.at[pl.ds(r0, rpt)], hsc.at[pl.ds(r0, rpt)])

        @pl.when(cid == 0)
        def _():
            pltpu.sync_copy(hs_hbm.at[pl.ds(r0, rpt)],
                            acc.at[pl.ds(r0, rpt)])

        @pl.when(cid != 0)
        def _():
            pltpu.sync_copy(zeros_hbm, acc.at[pl.ds(r0, rpt)])

        plsc.subcore_barrier()

        def body(j, carry):
            pltpu.async_copy(hsc.at[sidx.at[j]], rows, gsem).wait()
            pltpu.sync_copy(rows, acc.at[didx.at[j]], add=True)
            return carry

        lax.fori_loop(0, NCHUNK, body, 0, unroll=False)
        plsc.subcore_barrier()
        pltpu.sync_copy(acc.at[pl.ds(r0, rpt)],
                        out_hbm.at[cid, pl.ds(r0, rpt)])

    return msg_kernel


def _dis_from_deg(d0_ref, d1_ref):
    deg = (d0_ref[...] + d1_ref[...])[:, 0:1] + 1.0
    return lax.rsqrt(deg)


def _tc_first(x_ref, w_ref, d0_ref, d1_ref, o_ref):
    dis = _dis_from_deg(d0_ref, d1_ref)
    h = jnp.dot(x_ref[...], w_ref[...], preferred_element_type=jnp.float32)
    o_ref[...] = h * dis


def _tc_mid(a0_ref, a1_ref, d0_ref, d1_ref, w_ref, b_ref, o_ref):
    dis = _dis_from_deg(d0_ref, d1_ref)
    pre = (a0_ref[...] + a1_ref[...]) * dis + b_ref[...]
    act = jnp.maximum(pre, 0.0)
    h = jnp.dot(act, w_ref[...], preferred_element_type=jnp.float32)
    o_ref[...] = h * dis


def _tc_final(a0_ref, a1_ref, d0_ref, d1_ref, b_ref, batch_ref,
              lw_ref, lb_ref, o_ref):
    npad = a0_ref.shape[0]
    dis = _dis_from_deg(d0_ref, d1_ref)
    h3 = (a0_ref[...] + a1_ref[...]) * dis + b_ref[...]
    seg = lax.broadcasted_iota(jnp.int32, (1, _G), 1)
    m = (batch_ref[...] == seg).astype(jnp.float32)
    psum = lax.dot_general(m, h3, (((0,), (0,)), ((), ())),
                           preferred_element_type=jnp.float32)
    cnt = lax.dot_general(m, jnp.ones((npad, 1), jnp.float32),
                          (((0,), (0,)), ((), ())),
                          preferred_element_type=jnp.float32)
    pooled = psum / jnp.maximum(cnt, 1.0)
    o_ref[...] = (jnp.dot(pooled, lw_ref[...],
                          preferred_element_type=jnp.float32) + lb_ref[...])


def kernel(x, edge_index, batch, W1, b1, W2, b2, W3, b3, lin_W, lin_b):
    N, F = x.shape
    H = W1.shape[1]
    C = lin_W.shape[1]
    E = edge_index.shape[1]

    NCHUNK = -(-E // (_NW * _CHUNK))
    NCHUNK = -(-NCHUNK // _NB) * _NB
    EPAD = _NW * NCHUNK * _CHUNK
    NPAD = -(-(N + 1) // (_NS * 16)) * (_NS * 16)
    rpt = NPAD // _NS

    src = edge_index[0]
    dst = edge_index[1]
    epad = EPAD - E
    srcp = jnp.concatenate(
        [src, jnp.zeros((epad,), jnp.int32)]).reshape(_NW, NCHUNK, _CHUNK)
    dstp = jnp.concatenate(
        [dst, jnp.full((epad,), N, jnp.int32)]).reshape(_NW, NCHUNK, _CHUNK)
    xp = jnp.zeros((NPAD, F), jnp.float32).at[:N].set(x)
    zeros_h = jnp.zeros((rpt, H), jnp.float32)
    zeros_d = jnp.zeros((rpt, _DW), jnp.float32)
    ones_d = jnp.ones((_CHUNK, _DW), jnp.float32)
    batchp = jnp.concatenate(
        [batch, jnp.full((NPAD - N,), _G, jnp.int32)]).reshape(NPAD, 1)
    b1r = b1.reshape(1, H)
    b2r = b2.reshape(1, H)
    b3r = b3.reshape(1, H)
    lbr = lin_b.reshape(1, C)

    deg_kernel = _make_deg_kernel(NPAD, NCHUNK)
    msg_kernel = _make_msg_kernel(NPAD, H, NCHUNK)

    degp = deg_kernel(dstp, ones_d, zeros_d)
    d0, d1 = degp[0], degp[1]

    h1s = pl.pallas_call(
        _tc_first,
        out_shape=jax.ShapeDtypeStruct((NPAD, H), jnp.float32),
    )(xp, W1, d0, d1)

    acc1 = msg_kernel(h1s, srcp, dstp, zeros_h)
    h2s = pl.pallas_call(
        _tc_mid,
        out_shape=jax.ShapeDtypeStruct((NPAD, H), jnp.float32),
    )(acc1[0], acc1[1], d0, d1, W2, b1r)

    acc2 = msg_kernel(h2s, srcp, dstp, zeros_h)
    h3s = pl.pallas_call(
        _tc_mid,
        out_shape=jax.ShapeDtypeStruct((NPAD, H), jnp.float32),
    )(acc2[0], acc2[1], d0, d1, W3, b2r)

    acc3 = msg_kernel(h3s, srcp, dstp, zeros_h)
    logits = pl.pallas_call(
        _tc_final,
        out_shape=jax.ShapeDtypeStruct((_G, C), jnp.float32),
    )(acc3[0], acc3[1], d0, d1, b3r, batchp, lin_W, lbr)

    return logits


# speedup vs baseline: 2.1992x; 1.2171x over previous
import functools

import jax
import jax.numpy as jnp
from jax import lax
from jax.experimental import pallas as pl
from jax.experimental.pallas import tpu as pltpu
from jax.experimental.pallas import tpu_sc as plsc

_NC = 2
_NS = 16
_NW = _NC * _NS
_CHUNK = 128
_DW = 8
_G = 64


def _make_deg_kernel(NPAD, NCHUNK):
    rpt = NPAD // _NS
    mesh = plsc.VectorSubcoreMesh(core_axis_name="c", subcore_axis_name="s")

    @functools.partial(
        pl.kernel,
        out_type=jax.ShapeDtypeStruct((_NC, NPAD, _DW), jnp.float32),
        mesh=mesh,
        scratch_types=[
            pltpu.VMEM((NCHUNK, _CHUNK), jnp.int32),
            pltpu.VMEM((_CHUNK, _DW), jnp.float32),
            pltpu.VMEM_SHARED((NPAD, _DW), jnp.float32),
            pltpu.SemaphoreType.DMA,
        ],
        compiler_params=pltpu.CompilerParams(use_tc_tiling_on_sc=False),
    )
    def deg_kernel(dst_hbm, ones_hbm, zeros_hbm, out_hbm, didx, ones_v, acc,
                   sem):
        cid = lax.axis_index("c")
        sid = lax.axis_index("s")
        wid = sid * _NC + cid
        r0 = sid * rpt
        pltpu.sync_copy(zeros_hbm, acc.at[pl.ds(r0, rpt)])
        pltpu.sync_copy(ones_hbm, ones_v)
        pltpu.sync_copy(dst_hbm.at[wid], didx)
        plsc.subcore_barrier()

        def fire(j, carry):
            pltpu.async_copy(ones_v, acc.at[didx.at[j]], sem, add=True)
            return carry

        lax.fori_loop(0, NCHUNK, fire, 0)

        def drain(j, carry):
            pltpu.make_async_copy(ones_v, acc.at[didx.at[j]], sem).wait()
            return carry

        lax.fori_loop(0, NCHUNK, drain, 0)
        plsc.subcore_barrier()
        pltpu.sync_copy(acc.at[pl.ds(r0, rpt)],
                        out_hbm.at[cid, pl.ds(r0, rpt)])

    return deg_kernel


_NB = 4


def _make_msg_kernel(NPAD, H, NCHUNK):
    rpt = NPAD // _NS
    mesh = plsc.VectorSubcoreMesh(core_axis_name="c", subcore_axis_name="s")

    @functools.partial(
        pl.kernel,
        out_type=jax.ShapeDtypeStruct((_NC, NPAD, H), jnp.float32),
        mesh=mesh,
        scratch_types=[
            pltpu.VMEM((NCHUNK, _CHUNK), jnp.int32),
            pltpu.VMEM((NCHUNK, _CHUNK), jnp.int32),
            [pltpu.VMEM((_CHUNK, H), jnp.float32)] * 2,
            [pltpu.SemaphoreType.DMA] * 2,
            pltpu.SemaphoreType.DMA,
            pltpu.VMEM_SHARED((NPAD, H), jnp.float32),
            pltpu.VMEM_SHARED((NPAD, H), jnp.float32),
        ],
        compiler_params=pltpu.CompilerParams(use_tc_tiling_on_sc=False),
    )
    def msg_kernel(hs_hbm, src_hbm, dst_hbm, zeros_hbm, out_hbm,
                   sidx, didx, rows, ssems, gsem, acc, hsc):
        cid = lax.axis_index("c")
        sid = lax.axis_index("s")
        wid = sid * _NC + cid
        r0 = sid * rpt

        pltpu.sync_copy(src_hbm.at[wid], sidx)
        pltpu.sync_copy(dst_hbm.at[wid], didx)

        pltpu.sync_copy(hs_hbm.at[pl.ds(r0, rpt)], hsc.at[pl.ds(r0, rpt)])

        @pl.when(cid == 0)
        def _():
            pltpu.sync_copy(hs_hbm.at[pl.ds(r0, rpt)],
                            acc.at[pl.ds(r0, rpt)])

        @pl.when(cid != 0)
        def _():
            pltpu.sync_copy(zeros_hbm, acc.at[pl.ds(r0, rpt)])

        plsc.subcore_barrier()

        def gather(j, b):
            pltpu.async_copy(hsc.at[sidx.at[j]], rows[b], gsem).wait()

        def scatter_start(j, b):
            pltpu.async_copy(rows[b], acc.at[didx.at[j]], ssems[b],
                             add=True)

        def scatter_wait(j, b):
            pltpu.make_async_copy(rows[b], acc.at[didx.at[j]],
                                  ssems[b]).wait()

        for b in range(2):
            gather(b, b)
            scatter_start(b, b)

        def body(j, carry):
            for b in range(2):
                chunk = j + b
                scatter_wait(chunk - 2, b)
                gather(chunk, b)
                scatter_start(chunk, b)
            return carry

        lax.fori_loop(1, NCHUNK // 2, lambda i, c: body(i * 2, c), 0,
                      unroll=False)
        for b in range(2):
            scatter_wait(NCHUNK - 2 + b, b)
        plsc.subcore_barrier()
        pltpu.sync_copy(acc.at[pl.ds(r0, rpt)],
                        out_hbm.at[cid, pl.ds(r0, rpt)])

    return msg_kernel


def _dis_from_deg(d0_ref, d1_ref):
    deg = (d0_ref[...] + d1_ref[...])[:, 0:1] + 1.0
    return lax.rsqrt(deg)


def _tc_first(x_ref, w_ref, d0_ref, d1_ref, o_ref):
    dis = _dis_from_deg(d0_ref, d1_ref)
    h = jnp.dot(x_ref[...], w_ref[...], preferred_element_type=jnp.float32)
    o_ref[...] = h * dis


def _tc_mid(a0_ref, a1_ref, d0_ref, d1_ref, w_ref, b_ref, o_ref):
    dis = _dis_from_deg(d0_ref, d1_ref)
    pre = (a0_ref[...] + a1_ref[...]) * dis + b_ref[...]
    act = jnp.maximum(pre, 0.0)
    h = jnp.dot(act, w_ref[...], preferred_element_type=jnp.float32)
    o_ref[...] = h * dis


def _tc_final(a0_ref, a1_ref, d0_ref, d1_ref, b_ref, batch_ref,
              lw_ref, lb_ref, o_ref):
    npad = a0_ref.shape[0]
    dis = _dis_from_deg(d0_ref, d1_ref)
    h3 = (a0_ref[...] + a1_ref[...]) * dis + b_ref[...]
    seg = lax.broadcasted_iota(jnp.int32, (1, _G), 1)
    m = (batch_ref[...] == seg).astype(jnp.float32)
    psum = lax.dot_general(m, h3, (((0,), (0,)), ((), ())),
                           preferred_element_type=jnp.float32)
    cnt = lax.dot_general(m, jnp.ones((npad, 1), jnp.float32),
                          (((0,), (0,)), ((), ())),
                          preferred_element_type=jnp.float32)
    pooled = psum / jnp.maximum(cnt, 1.0)
    o_ref[...] = (jnp.dot(pooled, lw_ref[...],
                          preferred_element_type=jnp.float32) + lb_ref[...])


def kernel(x, edge_index, batch, W1, b1, W2, b2, W3, b3, lin_W, lin_b):
    N, F = x.shape
    H = W1.shape[1]
    C = lin_W.shape[1]
    E = edge_index.shape[1]

    NCHUNK = -(-E // (_NW * _CHUNK))
    NCHUNK = -(-NCHUNK // _NB) * _NB
    EPAD = _NW * NCHUNK * _CHUNK
    NPAD = -(-(N + 1) // (_NS * 16)) * (_NS * 16)
    rpt = NPAD // _NS

    src = edge_index[0]
    dst = edge_index[1]
    epad = EPAD - E
    srcp = jnp.concatenate(
        [src, jnp.zeros((epad,), jnp.int32)]).reshape(_NW, NCHUNK, _CHUNK)
    dstp = jnp.concatenate(
        [dst, jnp.full((epad,), N, jnp.int32)]).reshape(_NW, NCHUNK, _CHUNK)
    xp = jnp.zeros((NPAD, F), jnp.float32).at[:N].set(x)
    zeros_h = jnp.zeros((rpt, H), jnp.float32)
    zeros_d = jnp.zeros((rpt, _DW), jnp.float32)
    ones_d = jnp.ones((_CHUNK, _DW), jnp.float32)
    batchp = jnp.concatenate(
        [batch, jnp.full((NPAD - N,), _G, jnp.int32)]).reshape(NPAD, 1)
    b1r = b1.reshape(1, H)
    b2r = b2.reshape(1, H)
    b3r = b3.reshape(1, H)
    lbr = lin_b.reshape(1, C)

    deg_kernel = _make_deg_kernel(NPAD, NCHUNK)
    msg_kernel = _make_msg_kernel(NPAD, H, NCHUNK)

    degp = deg_kernel(dstp, ones_d, zeros_d)
    d0, d1 = degp[0], degp[1]

    h1s = pl.pallas_call(
        _tc_first,
        out_shape=jax.ShapeDtypeStruct((NPAD, H), jnp.float32),
    )(xp, W1, d0, d1)

    acc1 = msg_kernel(h1s, srcp, dstp, zeros_h)
    h2s = pl.pallas_call(
        _tc_mid,
        out_shape=jax.ShapeDtypeStruct((NPAD, H), jnp.float32),
    )(acc1[0], acc1[1], d0, d1, W2, b1r)

    acc2 = msg_kernel(h2s, srcp, dstp, zeros_h)
    h3s = pl.pallas_call(
        _tc_mid,
        out_shape=jax.ShapeDtypeStruct((NPAD, H), jnp.float32),
    )(acc2[0], acc2[1], d0, d1, W3, b2r)

    acc3 = msg_kernel(h3s, srcp, dstp, zeros_h)
    logits = pl.pallas_call(
        _tc_final,
        out_shape=jax.ShapeDtypeStruct((_G, C), jnp.float32),
    )(acc3[0], acc3[1], d0, d1, b3r, batchp, lin_W, lbr)

    return logits
